# Initial kernel scaffold; baseline (speedup 1.0000x reference)
import jax
import jax.numpy as jnp
from jax.experimental import pallas as pl


def kernel(fw_adjs, bw_adjs, features, emb, fw_Ws0, fw_bs0, fw_Wn0, fw_bn0, fw_Ws1, fw_bs1, fw_Wn1, fw_bn1, bw_Ws0, bw_bs0, bw_Wn0, bw_bn0, bw_Ws1, bw_bs1, bw_Wn1, bw_bn1):
    raise NotImplementedError("write your pallas kernel here")


# speedup vs baseline: 1.8290x; 1.8290x over previous
import functools

import jax
import jax.numpy as jnp
from jax import lax
from jax.experimental import pallas as pl
from jax.experimental.pallas import tpu as pltpu
from jax.experimental.pallas import tpu_sc as plsc

_NC = 2
_NS = 16
_NW = _NC * _NS
_D = 128
_DEG = 32
_G = 500


def _worker_id():
    return lax.axis_index("s") * _NC + lax.axis_index("c")


@functools.lru_cache(maxsize=None)
def _sc_gather(B):
    rows_per_worker = B // _NW
    n_chunk = rows_per_worker // 64

    def body(table_hbm, idx_hbm, out_hbm, idx_v, rows_v, sem):
        w = _worker_id()
        pltpu.sync_copy(idx_hbm.at[w], idx_v)
        base = w * rows_per_worker

        def chunk(j, carry):
            pltpu.async_copy(table_hbm.at[idx_v.at[j]], rows_v, sem).wait()
            pltpu.sync_copy(rows_v, out_hbm.at[pl.ds(base + j * 64, 64)])
            return carry

        lax.fori_loop(0, n_chunk, chunk, 0)

    return pl.kernel(
        body,
        out_type=jax.ShapeDtypeStruct((B, _D), jnp.float32),
        mesh=plsc.VectorSubcoreMesh(core_axis_name="c", subcore_axis_name="s"),
        scratch_types=[
            pltpu.VMEM((n_chunk, 64), jnp.int32),
            pltpu.VMEM((64, _D), jnp.float32),
            pltpu.SemaphoreType.DMA,
        ],
    )


@functools.lru_cache(maxsize=None)
def _sc_gather_sum(B, C):
    rows_per_worker = B // _NW
    n_groups = rows_per_worker // C
    n_dma = (C * _DEG) // 128
    idx_rows_per_worker = n_groups * n_dma

    def body(table_hbm, idx_hbm, out_hbm, idx_v, rows_v, acc_v, sem):
        w = _worker_id()
        pltpu.sync_copy(
            idx_hbm.at[pl.ds(w * idx_rows_per_worker, idx_rows_per_worker)],
            idx_v)
        out_base = w * rows_per_worker

        def group(g, carry):
            cps = [
                pltpu.async_copy(
                    table_hbm.at[idx_v.at[g * n_dma + t]],
                    rows_v.at[pl.ds(t * 128, 128)],
                    sem)
                for t in range(n_dma)
            ]
            for cp in cps:
                cp.wait()

            def reduce_one(c, carry2):
                for s in range(_D // 16):
                    vals = [rows_v[c * _DEG + k, pl.ds(s * 16, 16)]
                            for k in range(_DEG)]
                    while len(vals) > 1:
                        nxt = [vals[i] + vals[i + 1]
                               for i in range(0, len(vals) - 1, 2)]
                        if len(vals) % 2:
                            nxt.append(vals[-1])
                        vals = nxt
                    acc_v[c, pl.ds(s * 16, 16)] = vals[0]
                return carry2

            lax.fori_loop(0, C, reduce_one, 0)
            pltpu.sync_copy(acc_v, out_hbm.at[pl.ds(out_base + g * C, C)])
            return carry

        lax.fori_loop(0, n_groups, group, 0)

    return pl.kernel(
        body,
        out_type=jax.ShapeDtypeStruct((B, _D), jnp.float32),
        mesh=plsc.VectorSubcoreMesh(core_axis_name="c", subcore_axis_name="s"),
        scratch_types=[
            pltpu.VMEM((idx_rows_per_worker, 128), jnp.int32),
            pltpu.VMEM((C * _DEG, _D), jnp.float32),
            pltpu.VMEM((C, _D), jnp.float32),
            pltpu.SemaphoreType.DMA,
        ],
    )


def _k1_body(x_ref, Wn_ref, Ws_ref, bs_ref, y_ref, s_ref):
    x = x_ref[...]
    y_ref[0] = jnp.dot(x, Wn_ref[0], preferred_element_type=jnp.float32)
    s_ref[0] = (jnp.dot(x, Ws_ref[0], preferred_element_type=jnp.float32)
                + bs_ref[0, 0])


def _k2_body(s0_ref, g_ref, bn0_ref, Ws1_ref, bs1_ref, Wn1_ref,
             s1_ref, y1_ref):
    ha = jnp.maximum(s0_ref[0], 0.0)
    hb = jnp.maximum(g_ref[0] * (1.0 / _DEG) + bn0_ref[0, 0], 0.0)
    Ws1 = Ws1_ref[0]
    Wn1 = Wn1_ref[0]
    s1_ref[0] = (jnp.dot(ha, Ws1[:_D], preferred_element_type=jnp.float32)
                 + jnp.dot(hb, Ws1[_D:], preferred_element_type=jnp.float32)
                 + bs1_ref[0, 0])
    y1_ref[0] = (jnp.dot(ha, Wn1[:_D], preferred_element_type=jnp.float32)
                 + jnp.dot(hb, Wn1[_D:], preferred_element_type=jnp.float32))


def _k3_body(s1_ref, g_ref, bn1_ref, hid_ref, pool_ref):
    inv = 1.0 / _DEG
    a = jnp.maximum(s1_ref[0, 0], 0.0)
    b = jnp.maximum(g_ref[0, 0] * inv + bn1_ref[0, 0], 0.0)
    c = jnp.maximum(s1_ref[1, 0], 0.0)
    d = jnp.maximum(g_ref[1, 0] * inv + bn1_ref[1, 0], 0.0)
    hid = jnp.concatenate([a, b, c, d], axis=1)
    hid_ref[0] = hid
    pool_ref[0, 0] = jnp.max(hid, axis=0)


def kernel(fw_adjs, bw_adjs, features, emb,
           fw_Ws0, fw_bs0, fw_Wn0, fw_bn0, fw_Ws1, fw_bs1, fw_Wn1, fw_bn1,
           bw_Ws0, bw_bs0, bw_Wn0, bw_bn0, bw_Ws1, bw_bs1, bw_Wn1, bw_bn1):
    N = fw_adjs.shape[0]
    NP = -(-N // 512) * 512
    B = 2 * NP
    NG = N // _G

    feat_pad = jnp.concatenate(
        [features.astype(jnp.int32), jnp.zeros((NP - N,), jnp.int32)])
    pad = jnp.zeros((NP - N, _DEG), jnp.int32)
    fw_i = jnp.concatenate([fw_adjs.astype(jnp.int32), pad], axis=0)
    bw_i = jnp.concatenate([bw_adjs.astype(jnp.int32) + N, pad], axis=0)
    hop_idx = jnp.concatenate(
        [fw_i.reshape(-1), bw_i.reshape(-1)]).reshape(-1, 128)

    Wn0_s = jnp.stack([fw_Wn0, bw_Wn0])
    Ws0_s = jnp.stack([fw_Ws0, bw_Ws0])
    bs0_s = jnp.stack([fw_bs0, bw_bs0]).reshape(2, 1, _D)
    bn0_s = jnp.stack([fw_bn0, bw_bn0]).reshape(2, 1, _D)
    Ws1_s = jnp.stack([fw_Ws1, bw_Ws1])
    Wn1_s = jnp.stack([fw_Wn1, bw_Wn1])
    bs1_s = jnp.stack([fw_bs1, bw_bs1]).reshape(2, 1, _D)
    bn1_s = jnp.stack([fw_bn1, bw_bn1]).reshape(2, 1, _D)

    x = _sc_gather(NP)(emb, feat_pad.reshape(_NW, -1, 64))[:N]

    RB = 2000
    grid = (2, N // RB)
    w_spec = pl.BlockSpec((1, _D, _D), lambda d, i: (d, 0, 0))
    b_spec = pl.BlockSpec((1, 1, _D), lambda d, i: (d, 0, 0))
    r_spec = pl.BlockSpec((1, RB, _D), lambda d, i: (d, i, 0))
    y0, s0 = pl.pallas_call(
        _k1_body,
        grid=grid,
        in_specs=[pl.BlockSpec((RB, _D), lambda d, i: (i, 0)),
                  w_spec, w_spec, b_spec],
        out_specs=[r_spec, r_spec],
        out_shape=[jax.ShapeDtypeStruct((2, N, _D), jnp.float32),
                   jax.ShapeDtypeStruct((2, N, _D), jnp.float32)],
    )(x, Wn0_s, Ws0_s, bs0_s)

    g0 = _sc_gather_sum(B, 16)(y0.reshape(2 * N, _D), hop_idx)
    g0 = g0.reshape(2, NP, _D)[:, :N]

    w2_spec = pl.BlockSpec((1, 2 * _D, _D), lambda d, i: (d, 0, 0))
    s1, y1 = pl.pallas_call(
        _k2_body,
        grid=grid,
        in_specs=[r_spec, r_spec, b_spec, w2_spec, b_spec, w2_spec],
        out_specs=[r_spec, r_spec],
        out_shape=[jax.ShapeDtypeStruct((2, N, _D), jnp.float32),
                   jax.ShapeDtypeStruct((2, N, _D), jnp.float32)],
    )(s0, g0, bn0_s, Ws1_s, bs1_s, Wn1_s)

    g1 = _sc_gather_sum(B, 16)(y1.reshape(2 * N, _D), hop_idx)
    g1 = g1.reshape(2, NP, _D)[:, :N]

    pair_spec = pl.BlockSpec((2, 1, _G, _D), lambda g: (0, g, 0, 0))
    hidden, pooled = pl.pallas_call(
        _k3_body,
        grid=(NG,),
        in_specs=[pair_spec, pair_spec,
                  pl.BlockSpec((2, 1, _D), lambda g: (0, 0, 0))],
        out_specs=[pl.BlockSpec((1, _G, 4 * _D), lambda g: (g, 0, 0)),
                   pl.BlockSpec((1, 1, 4 * _D), lambda g: (g, 0, 0))],
        out_shape=[jax.ShapeDtypeStruct((NG, _G, 4 * _D), jnp.float32),
                   jax.ShapeDtypeStruct((NG, 1, 4 * _D), jnp.float32)],
    )(s1.reshape(2, NG, _G, _D), g1.reshape(2, NG, _G, _D), bn1_s)

    graph_embedding = pooled.reshape(NG, 4 * _D)
    return hidden, (graph_embedding, graph_embedding)


# speedup vs baseline: 2.0692x; 1.1313x over previous
import functools

import jax
import jax.numpy as jnp
from jax import lax
from jax.experimental import pallas as pl
from jax.experimental.pallas import tpu as pltpu
from jax.experimental.pallas import tpu_sc as plsc

_NC = 2
_NS = 16
_NW = _NC * _NS
_D = 128
_DEG = 32
_G = 500


def _worker_id():
    return lax.axis_index("s") * _NC + lax.axis_index("c")


@functools.lru_cache(maxsize=None)
def _sc_gather(B):
    rows_per_worker = B // _NW
    n_chunk = rows_per_worker // 64

    def body(table_hbm, idx_hbm, out_hbm, idx_v, rows_v, sem):
        w = _worker_id()
        pltpu.sync_copy(idx_hbm.at[w], idx_v)
        base = w * rows_per_worker

        def chunk(j, carry):
            pltpu.async_copy(table_hbm.at[idx_v.at[j]], rows_v, sem).wait()
            pltpu.sync_copy(rows_v, out_hbm.at[pl.ds(base + j * 64, 64)])
            return carry

        lax.fori_loop(0, n_chunk, chunk, 0)

    return pl.kernel(
        body,
        out_type=jax.ShapeDtypeStruct((B, _D), jnp.float32),
        mesh=plsc.VectorSubcoreMesh(core_axis_name="c", subcore_axis_name="s"),
        scratch_types=[
            pltpu.VMEM((n_chunk, 64), jnp.int32),
            pltpu.VMEM((64, _D), jnp.float32),
            pltpu.SemaphoreType.DMA,
        ],
    )


@functools.lru_cache(maxsize=None)
def _sc_gather_sum(B, C):
    rows_per_worker = B // _NW
    n_groups = rows_per_worker // C
    n_dma = (C * _DEG) // 128
    idx_rows_per_worker = n_groups * n_dma

    def body(table_hbm, idx_hbm, out_hbm, idx_v,
             rows_v, acc_v, gsem0, gsem1, osem0, osem1):
        gsem = (gsem0, gsem1)
        osem = (osem0, osem1)
        w = _worker_id()
        pltpu.sync_copy(
            idx_hbm.at[pl.ds(w * idx_rows_per_worker, idx_rows_per_worker)],
            idx_v)
        out_base = w * rows_per_worker

        def fire(g, b):
            for t in range(n_dma):
                pltpu.async_copy(
                    table_hbm.at[idx_v.at[g * n_dma + t]],
                    rows_v.at[b].at[pl.ds(t * 128, 128)],
                    gsem[b])

        def drain(b):
            for t in range(n_dma):
                pltpu.make_async_copy(
                    table_hbm.at[idx_v.at[t]],
                    rows_v.at[b].at[pl.ds(t * 128, 128)],
                    gsem[b]).wait()

        fire(0, 0)

        def pair(i, carry):
            for b in range(2):
                g = 2 * i + b
                nxt = g + 1

                @pl.when(nxt < n_groups)
                def _():
                    fire(nxt, 1 - b)

                drain(b)

                @pl.when(g >= 2)
                def _():
                    pltpu.make_async_copy(
                        acc_v.at[b], out_hbm.at[pl.ds(out_base, C)],
                        osem[b]).wait()

                def reduce_one(c, carry2):
                    for s in range(_D // 16):
                        vals = [rows_v[b, c * _DEG + k, pl.ds(s * 16, 16)]
                                for k in range(_DEG)]
                        while len(vals) > 1:
                            nxt2 = [vals[j] + vals[j + 1]
                                    for j in range(0, len(vals) - 1, 2)]
                            if len(vals) % 2:
                                nxt2.append(vals[-1])
                            vals = nxt2
                        acc_v[b, c, pl.ds(s * 16, 16)] = vals[0]
                    return carry2

                lax.fori_loop(0, C, reduce_one, 0)
                pltpu.async_copy(
                    acc_v.at[b], out_hbm.at[pl.ds(out_base + g * C, C)],
                    osem[b])
            return carry

        lax.fori_loop(0, n_groups // 2, pair, 0)
        for b in range(2):
            pltpu.make_async_copy(
                acc_v.at[b], out_hbm.at[pl.ds(out_base, C)], osem[b]).wait()

    return pl.kernel(
        body,
        out_type=jax.ShapeDtypeStruct((B, _D), jnp.float32),
        mesh=plsc.VectorSubcoreMesh(core_axis_name="c", subcore_axis_name="s"),
        scratch_types=[
            pltpu.VMEM((idx_rows_per_worker, 128), jnp.int32),
            pltpu.VMEM((2, C * _DEG, _D), jnp.float32),
            pltpu.VMEM((2, C, _D), jnp.float32),
            pltpu.SemaphoreType.DMA,
            pltpu.SemaphoreType.DMA,
            pltpu.SemaphoreType.DMA,
            pltpu.SemaphoreType.DMA,
        ],
    )


def _k1_body(x_ref, Wn_ref, Ws_ref, bs_ref, y_ref, s_ref):
    x = x_ref[...]
    y_ref[0] = jnp.dot(x, Wn_ref[0], preferred_element_type=jnp.float32)
    s_ref[0] = (jnp.dot(x, Ws_ref[0], preferred_element_type=jnp.float32)
                + bs_ref[0, 0])


def _k2_body(s0_ref, g_ref, bn0_ref, Ws1_ref, bs1_ref, Wn1_ref,
             s1_ref, y1_ref):
    ha = jnp.maximum(s0_ref[0], 0.0)
    hb = jnp.maximum(g_ref[0] * (1.0 / _DEG) + bn0_ref[0, 0], 0.0)
    Ws1 = Ws1_ref[0]
    Wn1 = Wn1_ref[0]
    s1_ref[0] = (jnp.dot(ha, Ws1[:_D], preferred_element_type=jnp.float32)
                 + jnp.dot(hb, Ws1[_D:], preferred_element_type=jnp.float32)
                 + bs1_ref[0, 0])
    y1_ref[0] = (jnp.dot(ha, Wn1[:_D], preferred_element_type=jnp.float32)
                 + jnp.dot(hb, Wn1[_D:], preferred_element_type=jnp.float32))


def _k3_body(s1_ref, g_ref, bn1_ref, hid_ref, pool_ref):
    inv = 1.0 / _DEG
    a = jnp.maximum(s1_ref[0, 0], 0.0)
    b = jnp.maximum(g_ref[0, 0] * inv + bn1_ref[0, 0], 0.0)
    c = jnp.maximum(s1_ref[1, 0], 0.0)
    d = jnp.maximum(g_ref[1, 0] * inv + bn1_ref[1, 0], 0.0)
    hid = jnp.concatenate([a, b, c, d], axis=1)
    hid_ref[0] = hid
    pool_ref[0, 0] = jnp.max(hid, axis=0)


def kernel(fw_adjs, bw_adjs, features, emb,
           fw_Ws0, fw_bs0, fw_Wn0, fw_bn0, fw_Ws1, fw_bs1, fw_Wn1, fw_bn1,
           bw_Ws0, bw_bs0, bw_Wn0, bw_bn0, bw_Ws1, bw_bs1, bw_Wn1, bw_bn1):
    N = fw_adjs.shape[0]
    NP = -(-N // 512) * 512
    B = 2 * NP
    NG = N // _G

    feat_pad = jnp.concatenate(
        [features.astype(jnp.int32), jnp.zeros((NP - N,), jnp.int32)])
    pad = jnp.zeros((NP - N, _DEG), jnp.int32)
    fw_i = jnp.concatenate([fw_adjs.astype(jnp.int32), pad], axis=0)
    bw_i = jnp.concatenate([bw_adjs.astype(jnp.int32) + N, pad], axis=0)
    hop_idx = jnp.concatenate(
        [fw_i.reshape(-1), bw_i.reshape(-1)]).reshape(-1, 128)

    Wn0_s = jnp.stack([fw_Wn0, bw_Wn0])
    Ws0_s = jnp.stack([fw_Ws0, bw_Ws0])
    bs0_s = jnp.stack([fw_bs0, bw_bs0]).reshape(2, 1, _D)
    bn0_s = jnp.stack([fw_bn0, bw_bn0]).reshape(2, 1, _D)
    Ws1_s = jnp.stack([fw_Ws1, bw_Ws1])
    Wn1_s = jnp.stack([fw_Wn1, bw_Wn1])
    bs1_s = jnp.stack([fw_bs1, bw_bs1]).reshape(2, 1, _D)
    bn1_s = jnp.stack([fw_bn1, bw_bn1]).reshape(2, 1, _D)

    x = _sc_gather(NP)(emb, feat_pad.reshape(_NW, -1, 64))[:N]

    RB = 2000
    grid = (2, N // RB)
    w_spec = pl.BlockSpec((1, _D, _D), lambda d, i: (d, 0, 0))
    b_spec = pl.BlockSpec((1, 1, _D), lambda d, i: (d, 0, 0))
    r_spec = pl.BlockSpec((1, RB, _D), lambda d, i: (d, i, 0))
    y0, s0 = pl.pallas_call(
        _k1_body,
        grid=grid,
        in_specs=[pl.BlockSpec((RB, _D), lambda d, i: (i, 0)),
                  w_spec, w_spec, b_spec],
        out_specs=[r_spec, r_spec],
        out_shape=[jax.ShapeDtypeStruct((2, N, _D), jnp.float32),
                   jax.ShapeDtypeStruct((2, N, _D), jnp.float32)],
    )(x, Wn0_s, Ws0_s, bs0_s)

    g0 = _sc_gather_sum(B, 8)(y0.reshape(2 * N, _D), hop_idx)
    g0 = g0.reshape(2, NP, _D)[:, :N]

    w2_spec = pl.BlockSpec((1, 2 * _D, _D), lambda d, i: (d, 0, 0))
    s1, y1 = pl.pallas_call(
        _k2_body,
        grid=grid,
        in_specs=[r_spec, r_spec, b_spec, w2_spec, b_spec, w2_spec],
        out_specs=[r_spec, r_spec],
        out_shape=[jax.ShapeDtypeStruct((2, N, _D), jnp.float32),
                   jax.ShapeDtypeStruct((2, N, _D), jnp.float32)],
    )(s0, g0, bn0_s, Ws1_s, bs1_s, Wn1_s)

    g1 = _sc_gather_sum(B, 8)(y1.reshape(2 * N, _D), hop_idx)
    g1 = g1.reshape(2, NP, _D)[:, :N]

    pair_spec = pl.BlockSpec((2, 1, _G, _D), lambda g: (0, g, 0, 0))
    hidden, pooled = pl.pallas_call(
        _k3_body,
        grid=(NG,),
        in_specs=[pair_spec, pair_spec,
                  pl.BlockSpec((2, 1, _D), lambda g: (0, 0, 0))],
        out_specs=[pl.BlockSpec((1, _G, 4 * _D), lambda g: (g, 0, 0)),
                   pl.BlockSpec((1, 1, 4 * _D), lambda g: (g, 0, 0))],
        out_shape=[jax.ShapeDtypeStruct((NG, _G, 4 * _D), jnp.float32),
                   jax.ShapeDtypeStruct((NG, 1, 4 * _D), jnp.float32)],
    )(s1.reshape(2, NG, _G, _D), g1.reshape(2, NG, _G, _D), bn1_s)

    graph_embedding = pooled.reshape(NG, 4 * _D)
    return hidden, (graph_embedding, graph_embedding)
